# baseline (speedup 1.0000x reference)
import functools
import math

import jax
import jax.numpy as jnp
from jax import lax
from jax.experimental import pallas as pl
from jax.experimental.pallas import tpu as pltpu
from jax.experimental.pallas import tpu_sc as plsc

_NUM_CORES = 2
_NUM_SUBCORES = 16
_NW = _NUM_CORES * _NUM_SUBCORES
_CHUNK = 256


def _table_body(tab_ref, w1_ref, b1_ref, w2_ref, b2_ref, out_ref):
    nb = tab_ref.shape[0]
    h = w2_ref.shape[0]
    hour = lax.broadcasted_iota(jnp.int32, (nb, h), 0).astype(jnp.float32)
    ang = 2.0 * math.pi * hour / 24.0
    s = jnp.sin(ang)
    c = jnp.cos(ang)
    hidden = jnp.maximum(s * w1_ref[0:1, :] + c * w1_ref[1:2, :] + b1_ref[:], 0.0)
    cont = jnp.dot(hidden, w2_ref[:], preferred_element_type=jnp.float32) + b2_ref[:]
    out_ref[:, : tab_ref.shape[1]] = tab_ref[:]
    out_ref[:, tab_ref.shape[1] :] = cont


def _build_table(hour_table, W1, b1, W2, b2):
    nb, e = hour_table.shape
    h = W2.shape[0]
    return pl.pallas_call(
        _table_body,
        out_shape=jax.ShapeDtypeStruct((nb, e + h), jnp.float32),
    )(hour_table, W1, b1.reshape(1, h), W2, b2.reshape(1, h))


_CB = 4


def _make_gather(nbatch, hist, d, nb, nbatch_sc):
    assert nbatch_sc % (_NW * _CB) == 0
    bpw = nbatch_sc // _NW
    rows = bpw * hist
    crows = _CB * hist
    nchunk = bpw // _CB
    assert nchunk >= 3 and nchunk % 2 == 0
    nfull = crows // 16
    tail = crows % 16
    mesh = plsc.VectorSubcoreMesh(core_axis_name="c", subcore_axis_name="s")

    @functools.partial(
        pl.kernel,
        mesh=mesh,
        compiler_params=pltpu.CompilerParams(needs_layout_passes=False),
        out_type=jax.ShapeDtypeStruct((nbatch, hist, d), jnp.float32),
        scratch_types=[
            pltpu.VMEM((rows,), jnp.int32),
            pltpu.VMEM((nb * d,), jnp.float32),
            pltpu.VMEM((_CB, hist, d), jnp.float32),
            pltpu.VMEM((_CB, hist, d), jnp.float32),
            pltpu.SemaphoreType.DMA,
            pltpu.SemaphoreType.DMA,
        ],
    )
    def gather_kernel(table_hbm, idx_hbm, out_hbm, idx_v, tab_v, buf0, buf1, w0, w1):
        wid = lax.axis_index("s") * _NUM_CORES + lax.axis_index("c")
        base_b = wid * bpw
        pltpu.sync_copy(table_hbm, tab_v)
        pltpu.sync_copy(idx_hbm.at[pl.ds(base_b * hist, rows)], idx_v)
        bufs = (buf0, buf1)
        wsems = (w0, w1)
        iota = lax.broadcasted_iota(jnp.int32, (16,), 0)

        def compute_chunk(c, buf):
            row0 = c * crows

            def group_at(off):
                iv = idx_v[pl.ds(row0 + off, 16)]
                ivs = iv * d
                rv = iota + off
                bv = rv // hist
                tv = rv - bv * hist

                @plsc.parallel_loop(0, d, unroll=16)
                def _k(k):
                    cv = jnp.bitwise_xor(iota, k)
                    vals = plsc.load_gather(tab_v, [ivs + cv])
                    plsc.store_scatter(buf, [bv, tv, cv], vals)

            def group(g, _):
                group_at(g * 16)
                return 0

            lax.fori_loop(0, nfull, group, 0)
            if tail:
                group_at(crows - 16)

        def wb_start(c, b):
            pltpu.async_copy(
                bufs[b], out_hbm.at[pl.ds(base_b + c * _CB, _CB)], wsems[b]
            )

        def wb_wait(b):
            pltpu.make_async_copy(
                bufs[b], out_hbm.at[pl.ds(0, _CB)], wsems[b]
            ).wait()

        for b in range(2):
            compute_chunk(b, bufs[b])
            wb_start(b, b)

        def body(p, _):
            for b in range(2):
                c = 2 * p + b
                wb_wait(b)
                compute_chunk(c, bufs[b])
                wb_start(c, b)
            return 0

        lax.fori_loop(1, nchunk // 2, body, 0)

        for b in range(2):
            wb_wait(b)

    return gather_kernel


_TB = 16


def _fill_body(alias_ref, h_ref, t_ref, o_ref):
    del alias_ref
    hb = h_ref[:]
    oh = (hb == lax.broadcasted_iota(jnp.int32, (hb.shape[0], 128), 1)).astype(
        jnp.float32
    )
    res = jnp.dot(oh, t_ref[:], preferred_element_type=jnp.float32)
    o_ref[:] = res.reshape(o_ref.shape)


def _tc_fill(nbatch, hist, d, start_b):
    rows = _TB * hist
    nblk = (nbatch - start_b) // _TB
    return pl.pallas_call(
        _fill_body,
        grid=(nblk,),
        in_specs=[
            pl.BlockSpec((_TB, hist, d), lambda i: (start_b // _TB + i, 0, 0)),
            pl.BlockSpec((rows, 1), lambda i: (start_b * hist // rows + i, 0)),
            pl.BlockSpec((128, d), lambda i: (0, 0)),
        ],
        out_specs=pl.BlockSpec((_TB, hist, d), lambda i: (start_b // _TB + i, 0, 0)),
        out_shape=jax.ShapeDtypeStruct((nbatch, hist, d), jnp.float32),
        input_output_aliases={0: 0},
    )


def kernel(hours, hour_table, W1, b1, W2, b2):
    table = _build_table(hour_table, W1, b1, W2, b2)
    nb, d = table.shape
    table_flat = table.reshape(-1)
    nbatch, hist = hours.shape
    flat = hours.reshape(-1)
    nbatch_sc = nbatch // 2
    out_sc = _make_gather(nbatch, hist, d, nb, nbatch_sc)(table_flat, flat)
    table_pad = jnp.zeros((128, d), jnp.float32).at[:nb].set(table)
    hours_col = flat.reshape(-1, 1)
    return _tc_fill(nbatch, hist, d, nbatch_sc)(out_sc, hours_col, table_pad)


# speedup vs baseline: 1.2625x; 1.2625x over previous
import functools
import math

import jax
import jax.numpy as jnp
from jax import lax
from jax.experimental import pallas as pl
from jax.experimental.pallas import tpu as pltpu
from jax.experimental.pallas import tpu_sc as plsc

_NUM_CORES = 2
_NUM_SUBCORES = 16
_NW = _NUM_CORES * _NUM_SUBCORES
_CHUNK = 256


def _table_body(tab_ref, w1_ref, b1_ref, w2_ref, b2_ref, out_ref):
    nb = tab_ref.shape[0]
    h = w2_ref.shape[0]
    hour = lax.broadcasted_iota(jnp.int32, (nb, h), 0).astype(jnp.float32)
    ang = 2.0 * math.pi * hour / 24.0
    s = jnp.sin(ang)
    c = jnp.cos(ang)
    hidden = jnp.maximum(s * w1_ref[0:1, :] + c * w1_ref[1:2, :] + b1_ref[:], 0.0)
    cont = jnp.dot(hidden, w2_ref[:], preferred_element_type=jnp.float32) + b2_ref[:]
    out_ref[:, : tab_ref.shape[1]] = tab_ref[:]
    out_ref[:, tab_ref.shape[1] :] = cont


def _build_table(hour_table, W1, b1, W2, b2):
    nb, e = hour_table.shape
    h = W2.shape[0]
    return pl.pallas_call(
        _table_body,
        out_shape=jax.ShapeDtypeStruct((nb, e + h), jnp.float32),
    )(hour_table, W1, b1.reshape(1, h), W2, b2.reshape(1, h))


_CB = 4


def _make_gather(nbatch, hist, d, nb):
    assert nbatch % (_NW * _CB) == 0
    bpw = nbatch // _NW
    rows = bpw * hist
    crows = _CB * hist
    nchunk = bpw // _CB
    assert nchunk >= 3 and nchunk % 2 == 0
    nfull = crows // 16
    tail = crows % 16
    mesh = plsc.VectorSubcoreMesh(core_axis_name="c", subcore_axis_name="s")

    @functools.partial(
        pl.kernel,
        mesh=mesh,
        compiler_params=pltpu.CompilerParams(needs_layout_passes=False),
        out_type=jax.ShapeDtypeStruct((nbatch, hist, d), jnp.float32),
        scratch_types=[
            pltpu.VMEM((rows,), jnp.int32),
            pltpu.VMEM((nb * d,), jnp.float32),
            pltpu.VMEM((_CB, hist, d), jnp.float32),
            pltpu.VMEM((_CB, hist, d), jnp.float32),
            pltpu.SemaphoreType.DMA,
            pltpu.SemaphoreType.DMA,
        ],
    )
    def gather_kernel(table_hbm, idx_hbm, out_hbm, idx_v, tab_v, buf0, buf1, w0, w1):
        wid = lax.axis_index("s") * _NUM_CORES + lax.axis_index("c")
        base_b = wid * bpw
        pltpu.sync_copy(table_hbm, tab_v)
        pltpu.sync_copy(idx_hbm.at[pl.ds(base_b * hist, rows)], idx_v)
        bufs = (buf0, buf1)
        wsems = (w0, w1)
        iota = lax.broadcasted_iota(jnp.int32, (16,), 0)

        def compute_chunk(c, buf):
            row0 = c * crows

            def group_at(off):
                iv = idx_v[pl.ds(row0 + off, 16)]
                ivs = iv * d
                rv = iota + off
                bv = rv // hist
                tv = rv - bv * hist

                @plsc.parallel_loop(0, d, unroll=16)
                def _k(k):
                    cv = jnp.bitwise_xor(iota, k)
                    vals = plsc.load_gather(tab_v, [ivs + cv])
                    plsc.store_scatter(buf, [bv, tv, cv], vals)

            def group(g, _):
                group_at(g * 16)
                return 0

            lax.fori_loop(0, nfull, group, 0)
            if tail:
                group_at(crows - 16)

        def wb_start(c, b):
            pltpu.async_copy(
                bufs[b], out_hbm.at[pl.ds(base_b + c * _CB, _CB)], wsems[b]
            )

        def wb_wait(b):
            pltpu.make_async_copy(
                bufs[b], out_hbm.at[pl.ds(0, _CB)], wsems[b]
            ).wait()

        for b in range(2):
            compute_chunk(b, bufs[b])
            wb_start(b, b)

        def body(p, _):
            for b in range(2):
                c = 2 * p + b
                wb_wait(b)
                compute_chunk(c, bufs[b])
                wb_start(c, b)
            return 0

        lax.fori_loop(1, nchunk // 2, body, 0)

        for b in range(2):
            wb_wait(b)

    return gather_kernel


def kernel(hours, hour_table, W1, b1, W2, b2):
    table = _build_table(hour_table, W1, b1, W2, b2)
    nb, d = table.shape
    table_flat = table.reshape(-1)
    nbatch, hist = hours.shape
    flat = hours.reshape(-1)
    return _make_gather(nbatch, hist, d, nb)(table_flat, flat)
